# baseline (speedup 1.0000x reference)
import functools

import jax
import jax.numpy as jnp
from jax import lax
from jax.experimental import pallas as pl
from jax.experimental.pallas import tpu as pltpu
from jax.experimental.pallas import tpu_sc as plsc

_NE = 1024
_D = 64
_P = 4096
_B = 8
_NW = 32
_BPW = (_B * _P) // _NW


def _tc_body(x_ref, e_ref, idx_ref, ep_ref, e2_ref):
    e = e_ref[...]

    @pl.when(pl.program_id(0) == 0)
    def _init():
        e2_ref[...] = jnp.sum(e * e, axis=1, keepdims=True)

    ep_ref[...] = jnp.concatenate(
        [e, jnp.zeros((_NE, 128 - _D), jnp.float32)], axis=1)

    x = x_ref[0]
    x2 = jnp.sum(x * x, axis=0, keepdims=True)
    e2 = e2_ref[...]
    mm2 = jax.lax.dot_general(e + e, x, (((1,), (0,)), ((), ())))
    dist = (x2 + e2) - mm2
    m = jnp.min(dist, axis=0, keepdims=True)
    jidx = jax.lax.broadcasted_iota(jnp.int32, (_NE, _P), 0).astype(jnp.float32)
    idxf = jnp.min(jnp.where(dist == m, jidx, float(_NE)), axis=0,
                   keepdims=True)
    idx_ref[0] = idxf.astype(jnp.int32)


def _sc_body(ep_hbm, idx_hbm, rows_hbm, idx_v, rows_v, sem):
    wid = lax.axis_index("s") * 2 + lax.axis_index("c")
    for k in range(2):
        base = wid * _BPW + k * (_BPW // 2)
        pltpu.sync_copy(idx_hbm.at[pl.ds(base, _BPW // 2)], idx_v)
        pltpu.async_copy(ep_hbm.at[idx_v], rows_v, sem).wait()
        pltpu.sync_copy(rows_v, rows_hbm.at[pl.ds(base, _BPW // 2)])


_sc_gather = functools.partial(
    pl.kernel,
    out_type=jax.ShapeDtypeStruct((_B * _P, 128), jnp.float32),
    mesh=plsc.VectorSubcoreMesh(core_axis_name="c", subcore_axis_name="s"),
    scratch_types=[
        pltpu.VMEM((_BPW // 2,), jnp.int32),
        pltpu.VMEM((_BPW // 2, 128), jnp.float32),
        pltpu.SemaphoreType.DMA,
    ],
)(_sc_body)


def _tr_body(r_ref, o_ref):
    o_ref[0] = jnp.transpose(r_ref[...][:, :_D], (1, 0))


def kernel(inputs, embedding):
    b, c, h, w = inputs.shape
    xf = inputs.reshape(b, c, h * w)
    npix = h * w
    idx = pl.pallas_call(
        _tc_body,
        grid=(b,),
        in_specs=[pl.BlockSpec((1, c, _P), lambda i: (i, 0, 0)),
                  pl.BlockSpec((_NE, _D), lambda i: (0, 0))],
        out_specs=[pl.BlockSpec((1, 1, _P), lambda i: (i, 0, 0)),
                   pl.BlockSpec((_NE, 128), lambda i: (0, 0))],
        out_shape=[jax.ShapeDtypeStruct((b, 1, npix), jnp.int32),
                   jax.ShapeDtypeStruct((_NE, 128), jnp.float32)],
        scratch_shapes=[pltpu.VMEM((_NE, 1), jnp.float32)],
    )(xf, embedding)
    idx, epad = idx
    rows = _sc_gather(epad, idx.reshape(-1))
    out = pl.pallas_call(
        _tr_body,
        grid=(b,),
        in_specs=[pl.BlockSpec((_P, 128), lambda i: (i, 0))],
        out_specs=pl.BlockSpec((1, _D, _P), lambda i: (i, 0, 0)),
        out_shape=jax.ShapeDtypeStruct((b, _D, npix), jnp.float32),
    )(rows)
    return out.reshape(b, c, h, w)


# speedup vs baseline: 1.5271x; 1.5271x over previous
import jax
import jax.numpy as jnp
from jax.experimental import pallas as pl
from jax.experimental.pallas import tpu as pltpu

_NE = 1024
_D = 64
_P = 4096


def _body(x_ref, e_ref, o_ref, e2_ref):
    e = e_ref[...]

    @pl.when((pl.program_id(0) == 0) & (pl.program_id(1) == 0))
    def _init():
        e2_ref[...] = jnp.sum(e * e, axis=1, keepdims=True)

    x = x_ref[0]
    x2 = jnp.sum(x * x, axis=0, keepdims=True)
    e2 = e2_ref[...]
    mm2 = jax.lax.dot_general(e + e, x, (((1,), (0,)), ((), ())))
    dist = (x2 + e2) - mm2
    m = jnp.min(dist, axis=0, keepdims=True)
    jidx = jax.lax.broadcasted_iota(jnp.int32, (_NE, _P), 0).astype(jnp.float32)
    idx = jnp.min(jnp.where(dist == m, jidx, float(_NE)), axis=0,
                  keepdims=True)
    oh = (jidx == idx).astype(jnp.float32)
    q = jax.lax.dot_general(e, oh, (((0,), (0,)), ((), ())))
    o_ref[0] = x + (q - x)


def kernel(inputs, embedding):
    b, c, h, w = inputs.shape
    xf = inputs.reshape(b, c, h * w)
    npix = h * w
    out = pl.pallas_call(
        _body,
        grid=(b, npix // _P),
        in_specs=[pl.BlockSpec((1, c, _P), lambda i, j: (i, 0, j)),
                  pl.BlockSpec((_NE, _D), lambda i, j: (0, 0))],
        out_specs=pl.BlockSpec((1, c, _P), lambda i, j: (i, 0, j)),
        out_shape=jax.ShapeDtypeStruct((b, c, npix), jnp.float32),
        scratch_shapes=[pltpu.VMEM((_NE, 1), jnp.float32)],
    )(xf, embedding)
    return out.reshape(b, c, h, w)
